# baseline (speedup 1.0000x reference)
import functools

import jax
import jax.numpy as jnp
from jax import lax
from jax.experimental import pallas as pl
from jax.experimental.pallas import tpu as pltpu
from jax.experimental.pallas import tpu_sc as plsc

_P = 8
_NTOK = 8192
_LDIM = 32
_NT = 5
_BLK = 256
_PADT = 8
_TBLW = 128


def _enc_conv(x, w, b, stride):
    y = lax.conv_general_dilated(
        x, w, window_strides=(stride, stride), padding=((1, 1), (1, 1)),
        dimension_numbers=('NCHW', 'OIHW', 'NCHW'))
    return y + b[None, :, None, None]


def _quantize_body(z_ref, zn_ref, cb_ref, idx_ref):
    z = z_ref[...]
    cb = cb_ref[...]
    cn = jnp.sum(cb * cb, axis=1)[None, :]
    s = lax.dot_general(z, cb, (((1,), (1,)), ((), ())),
                        precision=lax.Precision.HIGHEST)
    d2 = (zn_ref[...] + cn) - 2.0 * s
    d = jnp.sqrt(jnp.maximum(d2, 0.0))
    mv = jnp.min(d, axis=1, keepdims=True)
    ids = lax.broadcasted_iota(jnp.int32, d.shape, 1)
    cand = jnp.where(d == mv, ids, jnp.int32(2**30))
    idx_ref[...] = jnp.min(cand, axis=1)[:, None]


def _matcher_body(cb_ref, m1w_ref, m1b_ref, m2w_ref, m2b_ref, tbl_ref):
    cb = cb_ref[...]
    h1 = jnp.maximum(
        lax.dot_general(cb, m1w_ref[...], (((1,), (1,)), ((), ())),
                        precision=lax.Precision.HIGHEST) + m1b_ref[...], 0.0)
    logits = lax.dot_general(h1, m2w_ref[...], (((1,), (1,)), ((), ())),
                             precision=lax.Precision.HIGHEST) + m2b_ref[...]
    m = jnp.max(logits, axis=1, keepdims=True)
    e = jnp.exp(logits - m)
    probs = e / jnp.sum(e, axis=1, keepdims=True)
    ids = lax.broadcasted_iota(jnp.int32, logits.shape, 1)
    cand = jnp.where(logits == m, ids, jnp.int32(2**30))
    tok = jnp.min(cand, axis=1)
    tbl_ref[...] = jnp.concatenate(
        [probs, tok.astype(jnp.float32)[:, None],
         jnp.zeros((cb.shape[0], _TBLW - _PADT - 1), jnp.float32)], axis=1)


@functools.lru_cache(maxsize=None)
def _make_sc_gather(B, D):
    info = plsc.get_sparse_core_info()
    nc, ns = info.num_cores, info.num_subcores
    nw = nc * ns
    bpw = B // nw
    mesh = plsc.VectorSubcoreMesh(core_axis_name="c", subcore_axis_name="s")

    @functools.partial(
        pl.kernel, mesh=mesh,
        out_type=jax.ShapeDtypeStruct((B, D), jnp.float32),
        scratch_types=[
            pltpu.VMEM((bpw,), jnp.int32),
            pltpu.VMEM((bpw, D), jnp.float32),
            pltpu.SemaphoreType.DMA,
        ],
    )
    def gather_k(tbl_hbm, idx_hbm, out_hbm, idx_v, rows_v, sem):
        wid = lax.axis_index("s") * nc + lax.axis_index("c")
        base = wid * bpw
        pltpu.sync_copy(idx_hbm.at[pl.ds(base, bpw)], idx_v)
        pltpu.async_copy(tbl_hbm.at[idx_v], rows_v, sem).wait()
        pltpu.sync_copy(rows_v, out_hbm.at[pl.ds(base, bpw)])

    return gather_k


def _gather_rows(tbl, idx):
    return _make_sc_gather(idx.shape[0], tbl.shape[1])(tbl, idx)


def kernel(x, codebook, c1w, c1b, c2w, c2b, c3w, c3b, m1w, m1b, m2w, m2b):
    B, C, H, W = x.shape
    hp, wp = H // _P, W // _P
    patches = x.reshape(B, C, hp, _P, wp, _P)
    patches = jnp.transpose(patches, (0, 2, 4, 1, 3, 5))
    patches = patches.reshape(B * hp * wp, C, _P, _P)
    h = jax.nn.relu(_enc_conv(patches, c1w, c1b, 1))
    h = jax.nn.relu(_enc_conv(h, c2w, c2b, 2))
    h = _enc_conv(h, c3w, c3b, 2)
    z = jnp.mean(h, axis=(2, 3))
    zn = jnp.sum(z * z, axis=1, keepdims=True)

    n = z.shape[0]
    idx2 = pl.pallas_call(
        _quantize_body,
        grid=(n // _BLK,),
        in_specs=[
            pl.BlockSpec((_BLK, _LDIM), lambda i: (i, 0)),
            pl.BlockSpec((_BLK, 1), lambda i: (i, 0)),
            pl.BlockSpec((_NTOK, _LDIM), lambda i: (0, 0)),
        ],
        out_specs=pl.BlockSpec((_BLK, 1), lambda i: (i, 0)),
        out_shape=jax.ShapeDtypeStruct((n, 1), jnp.int32),
    )(z, zn, codebook)
    token_indices = idx2[:, 0]

    m2w_p = jnp.zeros((_PADT, 256), m2w.dtype).at[:_NT].set(m2w)
    m2b_p = jnp.full((1, _PADT), -1e30, m2b.dtype).at[0, :_NT].set(m2b)
    tbl = pl.pallas_call(
        _matcher_body,
        out_shape=jax.ShapeDtypeStruct((_NTOK, _TBLW), jnp.float32),
    )(codebook, m1w, m1b[None, :], m2w_p, m2b_p)

    g = jnp.take(tbl, token_indices, axis=0)
    ifs_tokens = g[:, _PADT].astype(jnp.int32)
    ifs_probs = g[:, :_NT]
    return (token_indices, ifs_tokens, ifs_probs)


# speedup vs baseline: 1.4397x; 1.4397x over previous
import jax
import jax.numpy as jnp
from jax import lax
from jax.experimental import pallas as pl

_P = 8
_NTOK = 8192
_NT = 5
_PADT = 8
_TBLW = 16


def _enc_conv(x, w, b, stride):
    y = lax.conv_general_dilated(
        x, w, window_strides=(stride, stride), padding=((1, 1), (1, 1)),
        dimension_numbers=('NCHW', 'OIHW', 'NCHW'))
    return y + b[None, :, None, None]


def _matcher_body(cb_ref, m1w_ref, m1b_ref, m2w_ref, m2b_ref, tbl_ref):
    qb = cb_ref[...].astype(jnp.bfloat16)
    dn = (((1,), (1,)), ((), ()))
    h1 = lax.dot_general(qb, m1w_ref[...].astype(jnp.bfloat16), dn,
                         preferred_element_type=jnp.float32)
    h1 = jnp.maximum(h1 + m1b_ref[...], 0.0).astype(jnp.bfloat16)
    logits = lax.dot_general(h1, m2w_ref[...].astype(jnp.bfloat16), dn,
                             preferred_element_type=jnp.float32) + m2b_ref[...]
    m = jnp.max(logits, axis=1, keepdims=True)
    e = jnp.exp(logits - m)
    probs = e / jnp.sum(e, axis=1, keepdims=True)
    ids = lax.broadcasted_iota(jnp.int32, logits.shape, 1)
    cand = jnp.where(logits == m, ids, jnp.int32(2**30))
    tok = jnp.min(cand, axis=1)
    tbl_ref[...] = jnp.concatenate(
        [probs, tok.astype(jnp.float32)[:, None],
         jnp.zeros((qb.shape[0], _TBLW - _PADT - 1), jnp.float32)], axis=1)


def kernel(x, codebook, c1w, c1b, c2w, c2b, c3w, c3b, m1w, m1b, m2w, m2b):
    B, C, H, W = x.shape
    hp, wp = H // _P, W // _P
    patches = x.reshape(B, C, hp, _P, wp, _P)
    patches = jnp.transpose(patches, (0, 2, 4, 1, 3, 5))
    patches = patches.reshape(B * hp * wp, C, _P, _P)
    h = jax.nn.relu(_enc_conv(patches, c1w, c1b, 1))
    h = jax.nn.relu(_enc_conv(h, c2w, c2b, 2))
    h = _enc_conv(h, c3w, c3b, 2)
    z = jnp.mean(h, axis=(2, 3))

    zn = jnp.sum(z * z, axis=1, keepdims=True)
    cn = jnp.sum(codebook * codebook, axis=1)[None, :]
    d2 = zn + cn - 2.0 * (z @ codebook.T)
    d = jnp.sqrt(jnp.maximum(d2, 0.0))
    token_indices = jnp.argmin(d, axis=1)

    m2w_p = jnp.zeros((_PADT, 256), m2w.dtype).at[:_NT].set(m2w)
    m2b_p = jnp.full((1, _PADT), -1e30, m2b.dtype).at[0, :_NT].set(m2b)
    tbl = pl.pallas_call(
        _matcher_body,
        out_shape=jax.ShapeDtypeStruct((_NTOK, _TBLW), jnp.float32),
    )(codebook, m1w, m1b[None, :], m2w_p, m2b_p)

    g = jnp.take(tbl, token_indices, axis=0)
    ifs_tokens = g[:, _PADT].astype(jnp.int32)
    ifs_probs = g[:, :_NT]
    return (token_indices, ifs_tokens, ifs_probs)
